# baseline (speedup 1.0000x reference)
import functools

import jax
import jax.numpy as jnp
from jax import lax
from jax.experimental import pallas as pl
from jax.experimental.pallas import tpu as pltpu
from jax.experimental.pallas import tpu_sc as plsc


def _sc_gather(table, idx):
    V, D = table.shape
    B = idx.shape[0]
    NW = 32
    bw = B // NW
    CH = max(d for d in range(8, 449, 8) if bw % d == 0)
    nch = bw // CH
    assert bw * NW == B and nch * CH == bw

    mesh = plsc.VectorSubcoreMesh(core_axis_name="c", subcore_axis_name="s")

    @functools.partial(
        pl.kernel,
        mesh=mesh,
        out_type=jax.ShapeDtypeStruct((B, D), jnp.float32),
        scratch_types=[
            pltpu.VMEM((bw,), jnp.int32),
            pltpu.VMEM((CH, D), jnp.float32),
            pltpu.VMEM((CH, D), jnp.float32),
            pltpu.SemaphoreType.DMA,
            pltpu.SemaphoreType.DMA,
            pltpu.SemaphoreType.DMA,
            pltpu.SemaphoreType.DMA,
        ],
    )
    def k(tab_hbm, idx_hbm, out_hbm, idx_v, rows_v0, rows_v1,
          sg0, sg1, sw0, sw1):
        wid = lax.axis_index("s") * 2 + lax.axis_index("c")
        base = wid * bw
        rows = (rows_v0, rows_v1)
        sg = (sg0, sg1)
        sw = (sw0, sw1)
        pltpu.sync_copy(idx_hbm.at[pl.ds(base, bw)], idx_v)

        def gat(c, b):
            return pltpu.make_async_copy(
                tab_hbm.at[idx_v.at[pl.ds(c * CH, CH)]], rows[b], sg[b])

        def wrb(c, b):
            return pltpu.make_async_copy(
                rows[b], out_hbm.at[pl.ds(base + c * CH, CH)], sw[b])

        gat(0, 0).start()
        for c in range(nch):
            b = c & 1
            bn = 1 - b
            gat(c, b).wait()
            if c + 1 < nch:
                if c >= 1:
                    wrb(c - 1, bn).wait()
                gat(c + 1, bn).start()
            wrb(c, b).start()
        if nch >= 2:
            wrb(nch - 2, nch & 1).wait()
        wrb(nch - 1, (nch - 1) & 1).wait()

    return k(table, idx)


def _pre_kernel(hv_ref, w1a_ref, w1c_ref, b1_ref, hv1_ref, p_ref):
    x = hv_ref[...]
    hv1_ref[...] = jnp.dot(x, w1a_ref[...], preferred_element_type=jnp.float32) + b1_ref[...]
    p_ref[...] = jnp.dot(x, w1c_ref[...], preferred_element_type=jnp.float32)


def _ln(x, s, b):
    mu = jnp.mean(x, axis=-1, keepdims=True)
    xc = x - mu
    var = jnp.mean(xc * xc, axis=-1, keepdims=True)
    return xc * jax.lax.rsqrt(var + 1e-5) * s + b


def _mpnn_kernel(BLK, K, D,
                 hv_ref, hv1_ref, he_ref, g_ref, w1b_ref, w2_ref, b2_ref,
                 w3_ref, b3_ref, n1s_ref, n1b_ref, wi_ref, bi_ref, wo_ref,
                 bo_ref, n2s_ref, n2b_ref, out_ref):
    he = he_ref[...]
    x = jnp.dot(he, w1b_ref[...], preferred_element_type=jnp.float32)
    x = x + g_ref[...]
    hv1 = hv1_ref[...]
    x = x + jnp.broadcast_to(hv1[:, None, :], (BLK, K, D)).reshape(BLK * K, D)
    m = jax.nn.gelu(x)
    m = jax.nn.gelu(jnp.dot(m, w2_ref[...], preferred_element_type=jnp.float32) + b2_ref[...])
    m = jnp.dot(m, w3_ref[...], preferred_element_type=jnp.float32) + b3_ref[...]
    dh = jnp.mean(m.reshape(BLK, K, D), axis=1)
    h = _ln(hv_ref[...] + dh, n1s_ref[...], n1b_ref[...])
    ff = jax.nn.gelu(jnp.dot(h, wi_ref[...], preferred_element_type=jnp.float32) + bi_ref[...])
    dh2 = jnp.dot(ff, wo_ref[...], preferred_element_type=jnp.float32) + bo_ref[...]
    out_ref[...] = _ln(h + dh2, n2s_ref[...], n2b_ref[...])


def _head_kernel(NN, H, RC, fp_ref, wc_ref, bf_ref, ba_ref, out_ref):
    chunks = []
    for j in range(NN // RC):
        acc = jnp.zeros((RC, 2 * H), jnp.float32)
        for t in range(9):
            xt = fp_ref[pl.ds(j * RC + t, RC), :]
            acc = acc + jnp.dot(xt, wc_ref[0, t], preferred_element_type=jnp.float32)
        chunks.append(acc)
    acc = jnp.concatenate(chunks, axis=0)
    o = acc[:, :H] + bf_ref[0]
    att = acc[:, H:] + ba_ref[0]
    mx = jnp.max(att, axis=0, keepdims=True)
    e = jnp.exp(att - mx)
    sm = e / jnp.sum(e, axis=0, keepdims=True)
    out_ref[...] = o * sm


def _mlp_kernel(w1_ref, b1_ref, w2_ref, b2_ref, emb_ref, out_ref):
    hdd = jax.nn.relu(jnp.dot(emb_ref[...], w1_ref[...], preferred_element_type=jnp.float32) + b1_ref[...])
    out_ref[...] = jnp.dot(hdd, w2_ref[...], preferred_element_type=jnp.float32) + b2_ref[...]


def kernel(h_V, h_E, mask, W1, b1, W2, b2, W3, b3, n1_s, n1_b, Wi, bi, Wo, bo,
           n2_s, n2_b, conv_f_w, conv_f_b, conv_a_w, conv_a_b, mlp_W1, mlp_b1,
           mlp_W2, mlp_b2, E_idx):
    N, D = h_V.shape
    K = h_E.shape[1]
    EMB = 2 * D
    H = D
    NL = mlp_b2.shape[0]
    HID = mlp_W1.shape[1]

    W1a, W1b, W1c = W1[:D], W1[D:2 * D], W1[2 * D:]
    r1 = lambda v: v.reshape(1, -1)

    PB = 2000
    hv1, P = pl.pallas_call(
        _pre_kernel,
        grid=(N // PB,),
        in_specs=[
            pl.BlockSpec((PB, D), lambda i: (i, 0)),
            pl.BlockSpec((D, D), lambda i: (0, 0)),
            pl.BlockSpec((D, D), lambda i: (0, 0)),
            pl.BlockSpec((1, D), lambda i: (0, 0)),
        ],
        out_specs=[
            pl.BlockSpec((PB, D), lambda i: (i, 0)),
            pl.BlockSpec((PB, D), lambda i: (i, 0)),
        ],
        out_shape=[
            jax.ShapeDtypeStruct((N, D), jnp.float32),
            jax.ShapeDtypeStruct((N, D), jnp.float32),
        ],
    )(h_V, W1a, W1c, r1(b1))

    G = _sc_gather(P, E_idx.reshape(-1).astype(jnp.int32))

    BLK = 400
    hE2 = h_E.reshape(N * K, D)
    full = lambda shape: pl.BlockSpec(shape, lambda i: tuple(0 for _ in shape))
    h2 = pl.pallas_call(
        functools.partial(_mpnn_kernel, BLK, K, D),
        grid=(N // BLK,),
        in_specs=[
            pl.BlockSpec((BLK, D), lambda i: (i, 0)),
            pl.BlockSpec((BLK, D), lambda i: (i, 0)),
            pl.BlockSpec((BLK * K, D), lambda i: (i, 0)),
            pl.BlockSpec((BLK * K, D), lambda i: (i, 0)),
            full((D, D)), full((D, D)), full((1, D)),
            full((D, D)), full((1, D)), full((1, D)), full((1, D)),
            full((D, 4 * D)), full((1, 4 * D)), full((4 * D, D)),
            full((1, D)), full((1, D)), full((1, D)),
        ],
        out_specs=pl.BlockSpec((BLK, D), lambda i: (i, 0)),
        out_shape=jax.ShapeDtypeStruct((N, D), jnp.float32),
    )(h_V, hv1, hE2, G, W1b, W2, r1(b2), W3, r1(b3), r1(n1_s),
      r1(n1_b), Wi, r1(bi), Wo, r1(bo), r1(n2_s), r1(n2_b))

    NP = 2
    H = EMB // NP
    feats_p = jnp.pad(jnp.concatenate([h_V, h2], axis=-1), ((4, 4), (0, 0)))
    wtf = conv_f_w.transpose(2, 1, 0)
    wta = conv_a_w.transpose(2, 1, 0)
    Wc = jnp.stack([
        jnp.concatenate([wtf[:, :, p * H:(p + 1) * H],
                         wta[:, :, p * H:(p + 1) * H]], axis=-1)
        for p in range(NP)
    ])
    bff = jnp.stack([conv_f_b[p * H:(p + 1) * H] for p in range(NP)]).reshape(NP, 1, H)
    ba = jnp.stack([conv_a_b[p * H:(p + 1) * H] for p in range(NP)]).reshape(NP, 1, H)

    emb = pl.pallas_call(
        functools.partial(_head_kernel, N, H, 2000),
        grid=(NP,),
        in_specs=[
            pl.BlockSpec((N + 8, EMB), lambda p: (0, 0)),
            pl.BlockSpec((1, 9, EMB, 2 * H), lambda p: (p, 0, 0, 0)),
            pl.BlockSpec((1, 1, H), lambda p: (p, 0, 0)),
            pl.BlockSpec((1, 1, H), lambda p: (p, 0, 0)),
        ],
        out_specs=pl.BlockSpec((N, H), lambda p: (0, p)),
        out_shape=jax.ShapeDtypeStruct((N, EMB), jnp.float32),
    )(feats_p, Wc, bff, ba)

    MB = 2000
    logits = pl.pallas_call(
        _mlp_kernel,
        grid=(N // MB,),
        in_specs=[
            pl.BlockSpec((EMB, HID), lambda i: (0, 0)),
            pl.BlockSpec((1, HID), lambda i: (0, 0)),
            pl.BlockSpec((HID, NL), lambda i: (0, 0)),
            pl.BlockSpec((1, NL), lambda i: (0, 0)),
            pl.BlockSpec((MB, EMB), lambda i: (i, 0)),
        ],
        out_specs=pl.BlockSpec((MB, NL), lambda i: (i, 0)),
        out_shape=jax.ShapeDtypeStruct((N, NL), jnp.float32),
    )(mlp_W1, r1(mlp_b1), mlp_W2, r1(mlp_b2), emb)

    return logits


# speedup vs baseline: 1.0272x; 1.0272x over previous
import functools

import jax
import jax.numpy as jnp
from jax import lax
from jax.experimental import pallas as pl
from jax.experimental.pallas import tpu as pltpu
from jax.experimental.pallas import tpu_sc as plsc


def _sc_gather(table, idx):
    V, D = table.shape
    B = idx.shape[0]
    NW = 32
    bw = B // NW
    CH = max(d for d in range(8, 449, 8) if bw % d == 0)
    nch = bw // CH
    assert bw * NW == B and nch * CH == bw

    mesh = plsc.VectorSubcoreMesh(core_axis_name="c", subcore_axis_name="s")

    @functools.partial(
        pl.kernel,
        mesh=mesh,
        out_type=jax.ShapeDtypeStruct((B, D), jnp.float32),
        scratch_types=[
            pltpu.VMEM((bw,), jnp.int32),
            pltpu.VMEM((CH, D), jnp.float32),
            pltpu.VMEM((CH, D), jnp.float32),
            pltpu.SemaphoreType.DMA,
            pltpu.SemaphoreType.DMA,
            pltpu.SemaphoreType.DMA,
            pltpu.SemaphoreType.DMA,
        ],
    )
    def k(tab_hbm, idx_hbm, out_hbm, idx_v, rows_v0, rows_v1,
          sg0, sg1, sw0, sw1):
        wid = lax.axis_index("s") * 2 + lax.axis_index("c")
        base = wid * bw
        rows = (rows_v0, rows_v1)
        sg = (sg0, sg1)
        sw = (sw0, sw1)
        pltpu.sync_copy(idx_hbm.at[pl.ds(base, bw)], idx_v)

        def gat(c, b):
            return pltpu.make_async_copy(
                tab_hbm.at[idx_v.at[pl.ds(c * CH, CH)]], rows[b], sg[b])

        def wrb(c, b):
            return pltpu.make_async_copy(
                rows[b], out_hbm.at[pl.ds(base + c * CH, CH)], sw[b])

        gat(0, 0).start()
        for c in range(nch):
            b = c & 1
            bn = 1 - b
            gat(c, b).wait()
            if c + 1 < nch:
                if c >= 1:
                    wrb(c - 1, bn).wait()
                gat(c + 1, bn).start()
            wrb(c, b).start()
        if nch >= 2:
            wrb(nch - 2, nch & 1).wait()
        wrb(nch - 1, (nch - 1) & 1).wait()

    return k(table, idx)


def _pre_kernel(hv_ref, w1a_ref, w1c_ref, b1_ref, hv1_ref, p_ref):
    x = hv_ref[...]
    hv1_ref[...] = jnp.dot(x, w1a_ref[...], preferred_element_type=jnp.float32) + b1_ref[...]
    p_ref[...] = jnp.dot(x, w1c_ref[...], preferred_element_type=jnp.float32)


def _ln(x, s, b):
    mu = jnp.mean(x, axis=-1, keepdims=True)
    xc = x - mu
    var = jnp.mean(xc * xc, axis=-1, keepdims=True)
    return xc * jax.lax.rsqrt(var + 1e-5) * s + b


def _mpnn_kernel(BLK, K, D,
                 hv_ref, hv1_ref, he_ref, g_ref, w1b_ref, w2_ref, b2_ref,
                 w3_ref, b3_ref, n1s_ref, n1b_ref, wi_ref, bi_ref, wo_ref,
                 bo_ref, n2s_ref, n2b_ref, out_ref):
    he = he_ref[...]
    x = jnp.dot(he, w1b_ref[...], preferred_element_type=jnp.float32)
    x = x + g_ref[...]
    hv1 = hv1_ref[...]
    x = x + jnp.broadcast_to(hv1[:, None, :], (BLK, K, D)).reshape(BLK * K, D)
    m = jax.nn.gelu(x)
    m = jax.nn.gelu(jnp.dot(m, w2_ref[...], preferred_element_type=jnp.float32) + b2_ref[...])
    m = jnp.dot(m, w3_ref[...], preferred_element_type=jnp.float32) + b3_ref[...]
    dh = jnp.mean(m.reshape(BLK, K, D), axis=1)
    h = _ln(hv_ref[...] + dh, n1s_ref[...], n1b_ref[...])
    ff = jax.nn.gelu(jnp.dot(h, wi_ref[...], preferred_element_type=jnp.float32) + bi_ref[...])
    dh2 = jnp.dot(ff, wo_ref[...], preferred_element_type=jnp.float32) + bo_ref[...]
    out_ref[...] = _ln(h + dh2, n2s_ref[...], n2b_ref[...])


def _head_kernel(NN, H, RC, hv_ref, h2_ref, wc_ref, bf_ref, ba_ref, out_ref):
    fp = jnp.pad(jnp.concatenate([hv_ref[...], h2_ref[...]], axis=-1),
                 ((4, 4), (0, 0)))
    chunks = []
    for j in range(NN // RC):
        acc = jnp.zeros((RC, 2 * H), jnp.float32)
        for t in range(9):
            xt = jax.lax.slice(fp, (j * RC + t, 0), (j * RC + t + RC, 2 * H))
            acc = acc + jnp.dot(xt, wc_ref[0, t], preferred_element_type=jnp.float32)
        chunks.append(acc)
    acc = jnp.concatenate(chunks, axis=0)
    o = acc[:, :H] + bf_ref[0]
    att = acc[:, H:] + ba_ref[0]
    mx = jnp.max(att, axis=0, keepdims=True)
    e = jnp.exp(att - mx)
    sm = e / jnp.sum(e, axis=0, keepdims=True)
    out_ref[...] = o * sm


def _mlp_kernel(w1_ref, b1_ref, w2_ref, b2_ref, emb_ref, out_ref):
    hdd = jax.nn.relu(jnp.dot(emb_ref[...], w1_ref[...], preferred_element_type=jnp.float32) + b1_ref[...])
    out_ref[...] = jnp.dot(hdd, w2_ref[...], preferred_element_type=jnp.float32) + b2_ref[...]


def kernel(h_V, h_E, mask, W1, b1, W2, b2, W3, b3, n1_s, n1_b, Wi, bi, Wo, bo,
           n2_s, n2_b, conv_f_w, conv_f_b, conv_a_w, conv_a_b, mlp_W1, mlp_b1,
           mlp_W2, mlp_b2, E_idx):
    N, D = h_V.shape
    K = h_E.shape[1]
    EMB = 2 * D
    H = D
    NL = mlp_b2.shape[0]
    HID = mlp_W1.shape[1]

    W1a, W1b, W1c = W1[:D], W1[D:2 * D], W1[2 * D:]
    r1 = lambda v: v.reshape(1, -1)

    PB = 2000
    hv1, P = pl.pallas_call(
        _pre_kernel,
        grid=(N // PB,),
        in_specs=[
            pl.BlockSpec((PB, D), lambda i: (i, 0)),
            pl.BlockSpec((D, D), lambda i: (0, 0)),
            pl.BlockSpec((D, D), lambda i: (0, 0)),
            pl.BlockSpec((1, D), lambda i: (0, 0)),
        ],
        out_specs=[
            pl.BlockSpec((PB, D), lambda i: (i, 0)),
            pl.BlockSpec((PB, D), lambda i: (i, 0)),
        ],
        out_shape=[
            jax.ShapeDtypeStruct((N, D), jnp.float32),
            jax.ShapeDtypeStruct((N, D), jnp.float32),
        ],
    )(h_V, W1a, W1c, r1(b1))

    G = _sc_gather(P, E_idx.reshape(-1).astype(jnp.int32))

    BLK = 400
    hE2 = h_E.reshape(N * K, D)
    full = lambda shape: pl.BlockSpec(shape, lambda i: tuple(0 for _ in shape))
    h2 = pl.pallas_call(
        functools.partial(_mpnn_kernel, BLK, K, D),
        grid=(N // BLK,),
        in_specs=[
            pl.BlockSpec((BLK, D), lambda i: (i, 0)),
            pl.BlockSpec((BLK, D), lambda i: (i, 0)),
            pl.BlockSpec((BLK * K, D), lambda i: (i, 0)),
            pl.BlockSpec((BLK * K, D), lambda i: (i, 0)),
            full((D, D)), full((D, D)), full((1, D)),
            full((D, D)), full((1, D)), full((1, D)), full((1, D)),
            full((D, 4 * D)), full((1, 4 * D)), full((4 * D, D)),
            full((1, D)), full((1, D)), full((1, D)),
        ],
        out_specs=pl.BlockSpec((BLK, D), lambda i: (i, 0)),
        out_shape=jax.ShapeDtypeStruct((N, D), jnp.float32),
    )(h_V, hv1, hE2, G, W1b, W2, r1(b2), W3, r1(b3), r1(n1_s),
      r1(n1_b), Wi, r1(bi), Wo, r1(bo), r1(n2_s), r1(n2_b))

    NP = 2
    H = EMB // NP
    wtf = conv_f_w.transpose(2, 1, 0)
    wta = conv_a_w.transpose(2, 1, 0)
    Wc = jnp.stack([
        jnp.concatenate([wtf[:, :, p * H:(p + 1) * H],
                         wta[:, :, p * H:(p + 1) * H]], axis=-1)
        for p in range(NP)
    ])
    bff = jnp.stack([conv_f_b[p * H:(p + 1) * H] for p in range(NP)]).reshape(NP, 1, H)
    ba = jnp.stack([conv_a_b[p * H:(p + 1) * H] for p in range(NP)]).reshape(NP, 1, H)

    emb = pl.pallas_call(
        functools.partial(_head_kernel, N, H, 2000),
        grid=(NP,),
        in_specs=[
            pl.BlockSpec((N, D), lambda p: (0, 0)),
            pl.BlockSpec((N, D), lambda p: (0, 0)),
            pl.BlockSpec((1, 9, EMB, 2 * H), lambda p: (p, 0, 0, 0)),
            pl.BlockSpec((1, 1, H), lambda p: (p, 0, 0)),
            pl.BlockSpec((1, 1, H), lambda p: (p, 0, 0)),
        ],
        out_specs=pl.BlockSpec((N, H), lambda p: (0, p)),
        out_shape=jax.ShapeDtypeStruct((N, EMB), jnp.float32),
    )(h_V, h2, Wc, bff, ba)

    MB = 2000
    logits = pl.pallas_call(
        _mlp_kernel,
        grid=(N // MB,),
        in_specs=[
            pl.BlockSpec((EMB, HID), lambda i: (0, 0)),
            pl.BlockSpec((1, HID), lambda i: (0, 0)),
            pl.BlockSpec((HID, NL), lambda i: (0, 0)),
            pl.BlockSpec((1, NL), lambda i: (0, 0)),
            pl.BlockSpec((MB, EMB), lambda i: (i, 0)),
        ],
        out_specs=pl.BlockSpec((MB, NL), lambda i: (i, 0)),
        out_shape=jax.ShapeDtypeStruct((N, NL), jnp.float32),
    )(mlp_W1, r1(mlp_b1), mlp_W2, r1(mlp_b2), emb)

    return logits
